# baseline (speedup 1.0000x reference)
import functools

import jax
import jax.numpy as jnp
from jax import lax
from jax.experimental import pallas as pl
from jax.experimental.pallas import tpu as pltpu
from jax.experimental.pallas import tpu_sc as plsc

E = 8
K = 2
D = 768
H = 3072
T = 2048
BLK = 256
NB = 23
NPAD = NB * BLK
BH = 512
NW = 32
_NEG = -1e30


def _router_body(x_ref, wg_ref, pos_ref, tok_ref, prob_ref, be_ref, aux_ref,
                 m_s, mc_s):
    x = x_ref[...]
    wg = wg_ref[...]
    scores = jax.lax.dot_general(
        x, wg, (((1,), (1,)), ((), ())), preferred_element_type=jnp.float32
    )
    idx = lax.broadcasted_iota(jnp.int32, scores.shape, 1)
    m0 = jnp.max(scores, axis=1, keepdims=True)
    i0 = jnp.min(jnp.where(scores >= m0, idx, E), axis=1, keepdims=True)
    oh0 = idx == i0
    s2 = jnp.where(oh0, _NEG, scores)
    m1 = jnp.max(s2, axis=1, keepdims=True)
    i1 = jnp.min(jnp.where(s2 >= m1, idx, E), axis=1, keepdims=True)
    oh1 = idx == i1
    p0 = 1.0 / (1.0 + jnp.exp(m1 - m0))
    p1 = 1.0 - p0
    g = jnp.exp(scores - m0)
    g = g / jnp.sum(g, axis=1, keepdims=True)
    avg_g = jnp.sum(g, axis=0) * (1.0 / T)
    counts_sel = jnp.sum(jnp.where(oh0 | oh1, 1.0, 0.0), axis=0)
    aux_ref[...] = jnp.reshape(
        E * jnp.sum(counts_sel * (1.0 / T) * avg_g), (1, 1))

    m_f = jnp.concatenate(
        [jnp.where(oh0, 1.0, 0.0), jnp.where(oh1, 1.0, 0.0)], axis=0
    )
    m_s[...] = m_f
    r_io = lax.broadcasted_iota(jnp.int32, (128, 128), 0)
    c_io = lax.broadcasted_iota(jnp.int32, (128, 128), 1)
    tril = jnp.where(r_io >= c_io, 1.0, 0.0)

    def cs_body(i, base):
        blk = m_s[pl.ds(i * 128, 128), :]
        c = jax.lax.dot_general(
            tril, blk, (((1,), (0,)), ((), ())),
            preferred_element_type=jnp.float32) + base
        mc_s[pl.ds(i * 128, 128), :] = c
        return lax.slice(c, (127, 0), (128, E))

    counts = lax.fori_loop(0, (2 * T) // 128, cs_body,
                           jnp.zeros((1, E), jnp.float32))

    nb = jnp.ceil(counts * (1.0 / BLK))
    r8 = lax.broadcasted_iota(jnp.int32, (E, E), 0)
    c8 = lax.broadcasted_iota(jnp.int32, (E, E), 1)
    upper_incl = jnp.where(r8 <= c8, 1.0, 0.0)
    incl_b = jax.lax.dot_general(
        nb, upper_incl, (((1,), (0,)), ((), ())),
        preferred_element_type=jnp.float32)
    excl_b = incl_b - nb
    seg_start = excl_b * float(BLK)

    mc = mc_s[...]
    pos_f = jnp.sum(m_s[...] * (mc - 1.0 + seg_start), axis=1, keepdims=True)
    pos_ref[...] = pos_f.astype(jnp.int32)

    t_io = lax.broadcasted_iota(jnp.int32, (T, 1), 0)
    tok_ref[...] = jnp.concatenate([t_io, t_io], axis=0)
    prob_ref[...] = jnp.concatenate([p0, p1], axis=0)

    b_io = lax.broadcasted_iota(jnp.int32, (NB, E), 0)
    excl_bi = excl_b.astype(jnp.int32)
    be = jnp.sum(jnp.where(b_io >= excl_bi, 1, 0), axis=1,
                 keepdims=True) - 1
    be_ref[...] = be


def _router(x_flat, Wg):
    return pl.pallas_call(
        _router_body,
        out_shape=(
            jax.ShapeDtypeStruct((2 * T, 1), jnp.int32),
            jax.ShapeDtypeStruct((2 * T, 1), jnp.int32),
            jax.ShapeDtypeStruct((2 * T, 1), jnp.float32),
            jax.ShapeDtypeStruct((NB, 1), jnp.int32),
            jax.ShapeDtypeStruct((1, 1), jnp.float32),
        ),
        in_specs=[
            pl.BlockSpec((T, D), lambda: (0, 0)),
            pl.BlockSpec((E, D), lambda: (0, 0)),
        ],
        out_specs=(
            pl.BlockSpec((2 * T, 1), lambda: (0, 0)),
            pl.BlockSpec((2 * T, 1), lambda: (0, 0)),
            pl.BlockSpec((2 * T, 1), lambda: (0, 0)),
            pl.BlockSpec((NB, 1), lambda: (0, 0)),
            pl.BlockSpec((1, 1), lambda: (0, 0)),
        ),
        scratch_shapes=[
            pltpu.VMEM((2 * T, E), jnp.float32),
            pltpu.VMEM((2 * T, E), jnp.float32),
        ],
    )(x_flat, Wg)


def _sc_mesh():
    return plsc.VectorSubcoreMesh(core_axis_name="c", subcore_axis_name="s")


@functools.cache
def _make_sc_dispatch():
    return functools.partial(
        pl.kernel, mesh=_sc_mesh(),
        out_type=(
            jax.ShapeDtypeStruct((NPAD,), jnp.float32),
            jax.ShapeDtypeStruct((NPAD,), jnp.int32),
        ),
        scratch_types=[
            pltpu.VMEM((2, 128), jnp.int32),
            pltpu.VMEM((2, 128), jnp.int32),
            pltpu.VMEM((2, 128), jnp.float32),
            pltpu.MemorySpace.VMEM_SHARED((NPAD,), jnp.int32),
            pltpu.MemorySpace.VMEM_SHARED((NPAD,), jnp.float32),
        ],
    )(_sc_dispatch_body)


def _sc_dispatch(pos2, tok2, prob2, zi, zf):
    return _make_sc_dispatch()(pos2, tok2, prob2, zi, zf)


def _sc_dispatch_body(pos_hbm, tok_hbm, prob_hbm, zi_hbm, zf_hbm,
                      wslot_hbm, rowtok_hbm,
                      idx_v, tval_v, pval_v, shtok, shw):
    c = lax.axis_index("c")
    s = lax.axis_index("s")
    wid = s * 2 + c

    @pl.when(s == 0)
    def _zero():
        pltpu.sync_copy(zi_hbm, shtok)
        pltpu.sync_copy(zf_hbm, shw)

    plsc.subcore_barrier()
    pltpu.sync_copy(pos_hbm.at[pl.ds(s * 2, 2)], idx_v)
    pltpu.sync_copy(tok_hbm.at[pl.ds(s * 2, 2)], tval_v)
    pltpu.sync_copy(prob_hbm.at[pl.ds(s * 2, 2)], pval_v)
    for j in range(2):
        pltpu.sync_copy(tval_v.at[j], shtok.at[idx_v.at[j]], add=True)
        pltpu.sync_copy(pval_v.at[j], shw.at[idx_v.at[j]], add=True)
    plsc.subcore_barrier()

    @pl.when(wid == 0)
    def _tables_out():
        pltpu.sync_copy(shw, wslot_hbm)
        pltpu.sync_copy(shtok, rowtok_hbm)


_RPW = NPAD // NW
_GC = (64, 64, 56)


@functools.cache
def _make_sc_gather():
    return functools.partial(
        pl.kernel, mesh=_sc_mesh(),
        out_type=jax.ShapeDtypeStruct((NPAD, D), jnp.float32),
        scratch_types=[
            pltpu.VMEM((64,), jnp.int32),
            pltpu.VMEM((64,), jnp.int32),
            pltpu.VMEM((56,), jnp.int32),
            pltpu.VMEM((64, D), jnp.float32),
            pltpu.VMEM((64, D), jnp.float32),
            pltpu.SemaphoreType.DMA,
            pltpu.SemaphoreType.DMA,
            pltpu.SemaphoreType.DMA,
            pltpu.SemaphoreType.DMA,
        ],
    )(_sc_gather_body)


def _sc_gather(rowtok, x_flat):
    return _make_sc_gather()(rowtok, x_flat)


def _sc_gather_body(rowtok_hbm, x_hbm, xs_hbm,
                    gidx0, gidx1, gidx2, buf0, buf1, gsem0, gsem1,
                    wsem0, wsem1):
    wid = lax.axis_index("s") * 2 + lax.axis_index("c")
    base = wid * _RPW
    gidx = (gidx0, gidx1, gidx2)
    bufs = (buf0, buf1, buf0)
    gsems = (gsem0, gsem1, gsem0)
    offs = (0, 64, 128)
    for i in range(3):
        pltpu.sync_copy(rowtok_hbm.at[pl.ds(base + offs[i], _GC[i])],
                        gidx[i])
    for i in range(3):
        b = bufs[i]
        dst = b if _GC[i] == 64 else b.at[pl.ds(0, _GC[i])]
        pltpu.async_copy(x_hbm.at[gidx[i]], dst, gsems[i]).wait()
        pltpu.sync_copy(dst, xs_hbm.at[pl.ds(base + offs[i], _GC[i])])


def _gmm_body(be_ref, xs_ref, ws_ref, w1_ref, b1_ref, w2_ref, b2_ref,
              w3_ref, b3_ref, o_ref):
    h = pl.program_id(1)
    x = xs_ref[...].astype(jnp.bfloat16)
    w1 = w1_ref[0].astype(jnp.bfloat16)
    w2 = w2_ref[0].astype(jnp.bfloat16)
    w3 = w3_ref[0].astype(jnp.bfloat16)
    a = jax.lax.dot_general(
        x, w1, (((1,), (1,)), ((), ())), preferred_element_type=jnp.float32
    ) + b1_ref[0]
    b = jax.lax.dot_general(
        x, w2, (((1,), (1,)), ((), ())), preferred_element_type=jnp.float32
    ) + b2_ref[0]
    ws = ws_ref[...]
    hact = ((a * jax.nn.sigmoid(a) * b) * ws).astype(jnp.bfloat16)
    y = jax.lax.dot_general(
        hact, w3, (((1,), (1,)), ((), ())), preferred_element_type=jnp.float32
    )

    @pl.when(h == 0)
    def _first():
        o_ref[...] = y + ws * b3_ref[0]

    @pl.when(h != 0)
    def _rest():
        o_ref[...] += y


def _gmm(be, xs, ws, W1, b1, W2, b2, W3, b3):
    grid_spec = pltpu.PrefetchScalarGridSpec(
        num_scalar_prefetch=1,
        grid=(NB, H // BH),
        in_specs=[
            pl.BlockSpec((BLK, D), lambda b, h, be_r: (b, 0)),
            pl.BlockSpec((BLK, 1), lambda b, h, be_r: (b, 0)),
            pl.BlockSpec((1, BH, D), lambda b, h, be_r: (be_r[b], h, 0)),
            pl.BlockSpec((1, 1, BH), lambda b, h, be_r: (be_r[b], 0, h)),
            pl.BlockSpec((1, BH, D), lambda b, h, be_r: (be_r[b], h, 0)),
            pl.BlockSpec((1, 1, BH), lambda b, h, be_r: (be_r[b], 0, h)),
            pl.BlockSpec((1, D, BH), lambda b, h, be_r: (be_r[b], 0, h)),
            pl.BlockSpec((1, 1, D), lambda b, h, be_r: (be_r[b], 0, 0)),
        ],
        out_specs=pl.BlockSpec((BLK, D), lambda b, h, be_r: (b, 0)),
    )
    return pl.pallas_call(
        _gmm_body,
        grid_spec=grid_spec,
        out_shape=jax.ShapeDtypeStruct((NPAD, D), jnp.float32),
        compiler_params=pltpu.CompilerParams(
            dimension_semantics=("arbitrary", "arbitrary"),
        ),
    )(be, xs, ws, W1, b1.reshape(E, 1, H), W2, b2.reshape(E, 1, H), W3,
      b3.reshape(E, 1, D))


_TPW = T // NW


@functools.cache
def _make_sc_combine():
    return functools.partial(
        pl.kernel, mesh=_sc_mesh(),
        out_type=(
            jax.ShapeDtypeStruct((T, D), jnp.float32),
            jax.ShapeDtypeStruct((T, D), jnp.float32),
        ),
        scratch_types=[
            pltpu.VMEM((_TPW,), jnp.int32),
            pltpu.VMEM((_TPW,), jnp.int32),
            pltpu.VMEM((_TPW, D), jnp.float32),
            pltpu.SemaphoreType.DMA,
        ],
    )(_sc_combine_body)


def _sc_combine(pos1d, ys):
    return _make_sc_combine()(pos1d, ys)


def _sc_combine_body(pos_hbm, ys_hbm, g0_hbm, g1_hbm,
                     idx0, idx1, rows_v, sem):
    wid = lax.axis_index("s") * 2 + lax.axis_index("c")
    base = wid * _TPW
    pltpu.sync_copy(pos_hbm.at[pl.ds(base, _TPW)], idx0)
    pltpu.sync_copy(pos_hbm.at[pl.ds(T + base, _TPW)], idx1)
    pltpu.async_copy(ys_hbm.at[idx0], rows_v, sem).wait()
    pltpu.sync_copy(rows_v, g0_hbm.at[pl.ds(base, _TPW)])
    pltpu.async_copy(ys_hbm.at[idx1], rows_v, sem).wait()
    pltpu.sync_copy(rows_v, g1_hbm.at[pl.ds(base, _TPW)])


def _add_body(a_ref, b_ref, o_ref):
    o_ref[...] = a_ref[...] + b_ref[...]


def _final_add(g0, g1):
    return pl.pallas_call(
        _add_body,
        grid=(4,),
        in_specs=[
            pl.BlockSpec((T // 4, D), lambda i: (i, 0)),
            pl.BlockSpec((T // 4, D), lambda i: (i, 0)),
        ],
        out_specs=pl.BlockSpec((T // 4, D), lambda i: (i, 0)),
        out_shape=jax.ShapeDtypeStruct((T, D), jnp.float32),
    )(g0, g1)


def kernel(x, Wg, W1, b1, W2, b2, W3, b3):
    B, S, _ = x.shape
    x_flat = x.reshape(T, D)

    pos, tok, prob, be, aux = _router(x_flat, Wg)
    wslot, rowtok = _sc_dispatch(
        pos.reshape(32, 128), tok.reshape(32, 128), prob.reshape(32, 128),
        jnp.zeros((NPAD,), jnp.int32), jnp.zeros((NPAD,), jnp.float32))
    xs = _sc_gather(rowtok, x_flat)
    ys = _gmm(be.reshape(NB), xs, wslot.reshape(NPAD, 1),
              W1, b1, W2, b2, W3, b3)
    g0, g1 = _sc_combine(pos.reshape(2 * T), ys)
    out = _final_add(g0, g1)
    return out.reshape(B, S, D), aux[0, 0]


# speedup vs baseline: 1.2419x; 1.2419x over previous
import functools

import jax
import jax.numpy as jnp
from jax import lax
from jax.experimental import pallas as pl
from jax.experimental.pallas import tpu as pltpu
from jax.experimental.pallas import tpu_sc as plsc

E = 8
K = 2
D = 768
H = 3072
T = 2048
BLK = 256
NB = 23
NPAD = NB * BLK
BH = 512
NW = 32
_NEG = -1e30


def _router_body(x_ref, wg_ref, pos_ref, tok_ref, prob_ref, be_ref, aux_ref,
                 m_s, mc_s):
    x = x_ref[...]
    wg = wg_ref[...]
    scores = jax.lax.dot_general(
        x, wg, (((1,), (1,)), ((), ())), preferred_element_type=jnp.float32
    )
    idx = lax.broadcasted_iota(jnp.int32, scores.shape, 1)
    m0 = jnp.max(scores, axis=1, keepdims=True)
    i0 = jnp.min(jnp.where(scores >= m0, idx, E), axis=1, keepdims=True)
    oh0 = idx == i0
    s2 = jnp.where(oh0, _NEG, scores)
    m1 = jnp.max(s2, axis=1, keepdims=True)
    i1 = jnp.min(jnp.where(s2 >= m1, idx, E), axis=1, keepdims=True)
    oh1 = idx == i1
    p0 = 1.0 / (1.0 + jnp.exp(m1 - m0))
    p1 = 1.0 - p0
    g = jnp.exp(scores - m0)
    g = g / jnp.sum(g, axis=1, keepdims=True)
    avg_g = jnp.sum(g, axis=0) * (1.0 / T)
    counts_sel = jnp.sum(jnp.where(oh0 | oh1, 1.0, 0.0), axis=0)
    aux_ref[...] = jnp.reshape(
        E * jnp.sum(counts_sel * (1.0 / T) * avg_g), (1, 1))

    m_f = jnp.concatenate(
        [jnp.where(oh0, 1.0, 0.0), jnp.where(oh1, 1.0, 0.0)], axis=0
    )
    m_s[...] = m_f
    r_io = lax.broadcasted_iota(jnp.int32, (128, 128), 0)
    c_io = lax.broadcasted_iota(jnp.int32, (128, 128), 1)
    tril = jnp.where(r_io >= c_io, 1.0, 0.0)

    def cs_body(i, base):
        blk = m_s[pl.ds(i * 128, 128), :]
        c = jax.lax.dot_general(
            tril, blk, (((1,), (0,)), ((), ())),
            preferred_element_type=jnp.float32) + base
        mc_s[pl.ds(i * 128, 128), :] = c
        return lax.slice(c, (127, 0), (128, E))

    counts = lax.fori_loop(0, (2 * T) // 128, cs_body,
                           jnp.zeros((1, E), jnp.float32))

    nb = jnp.ceil(counts * (1.0 / BLK))
    r8 = lax.broadcasted_iota(jnp.int32, (E, E), 0)
    c8 = lax.broadcasted_iota(jnp.int32, (E, E), 1)
    upper_incl = jnp.where(r8 <= c8, 1.0, 0.0)
    incl_b = jax.lax.dot_general(
        nb, upper_incl, (((1,), (0,)), ((), ())),
        preferred_element_type=jnp.float32)
    excl_b = incl_b - nb
    seg_start = excl_b * float(BLK)

    mc = mc_s[...]
    pos_f = jnp.sum(m_s[...] * (mc - 1.0 + seg_start), axis=1, keepdims=True)
    pos_ref[...] = pos_f.astype(jnp.int32)

    t_io = lax.broadcasted_iota(jnp.int32, (T, 1), 0)
    tok_all = jnp.concatenate([t_io, t_io], axis=0)
    pos_i = pos_f.astype(jnp.int32)
    init_at_pos = lax.rem(pos_i * 3 + 128, T)
    tok_ref[...] = tok_all - init_at_pos
    prob_ref[...] = jnp.concatenate([p0, p1], axis=0)

    b_io = lax.broadcasted_iota(jnp.int32, (NB, E), 0)
    excl_bi = excl_b.astype(jnp.int32)
    be = jnp.sum(jnp.where(b_io >= excl_bi, 1, 0), axis=1,
                 keepdims=True) - 1
    be_ref[...] = be


def _router(x_flat, Wg):
    return pl.pallas_call(
        _router_body,
        out_shape=(
            jax.ShapeDtypeStruct((2 * T, 1), jnp.int32),
            jax.ShapeDtypeStruct((2 * T, 1), jnp.int32),
            jax.ShapeDtypeStruct((2 * T, 1), jnp.float32),
            jax.ShapeDtypeStruct((NB, 1), jnp.int32),
            jax.ShapeDtypeStruct((1, 1), jnp.float32),
        ),
        in_specs=[
            pl.BlockSpec((T, D), lambda: (0, 0)),
            pl.BlockSpec((E, D), lambda: (0, 0)),
        ],
        out_specs=(
            pl.BlockSpec((2 * T, 1), lambda: (0, 0)),
            pl.BlockSpec((2 * T, 1), lambda: (0, 0)),
            pl.BlockSpec((2 * T, 1), lambda: (0, 0)),
            pl.BlockSpec((NB, 1), lambda: (0, 0)),
            pl.BlockSpec((1, 1), lambda: (0, 0)),
        ),
        scratch_shapes=[
            pltpu.VMEM((2 * T, E), jnp.float32),
            pltpu.VMEM((2 * T, E), jnp.float32),
        ],
    )(x_flat, Wg)


def _sc_mesh():
    return plsc.VectorSubcoreMesh(core_axis_name="c", subcore_axis_name="s")


@functools.cache
def _make_sc_dispatch():
    return functools.partial(
        pl.kernel, mesh=_sc_mesh(),
        out_type=(
            jax.ShapeDtypeStruct((NPAD,), jnp.float32),
            jax.ShapeDtypeStruct((NPAD,), jnp.int32),
        ),
        scratch_types=[
            pltpu.VMEM((2, 128), jnp.int32),
            pltpu.VMEM((2, 128), jnp.int32),
            pltpu.VMEM((2, 128), jnp.float32),
            pltpu.MemorySpace.VMEM_SHARED((NPAD,), jnp.int32),
            pltpu.MemorySpace.VMEM_SHARED((NPAD,), jnp.float32),
        ],
    )(_sc_dispatch_body)


def _sc_dispatch(pos2, tok2, prob2, zi, zf):
    return _make_sc_dispatch()(pos2, tok2, prob2, zi, zf)


def _sc_dispatch_body(pos_hbm, tok_hbm, prob_hbm, zi_hbm, zf_hbm,
                      wslot_hbm, rowtok_hbm,
                      idx_v, tval_v, pval_v, shtok, shw):
    c = lax.axis_index("c")
    s = lax.axis_index("s")
    wid = s * 2 + c

    @pl.when(s == 0)
    def _zero():
        pltpu.sync_copy(zi_hbm, shtok)
        pltpu.sync_copy(zf_hbm, shw)

    plsc.subcore_barrier()
    pltpu.sync_copy(pos_hbm.at[pl.ds(s * 2, 2)], idx_v)
    pltpu.sync_copy(tok_hbm.at[pl.ds(s * 2, 2)], tval_v)
    pltpu.sync_copy(prob_hbm.at[pl.ds(s * 2, 2)], pval_v)
    for j in range(2):
        pltpu.sync_copy(tval_v.at[j], shtok.at[idx_v.at[j]], add=True)
        pltpu.sync_copy(pval_v.at[j], shw.at[idx_v.at[j]], add=True)
    plsc.subcore_barrier()

    @pl.when(wid == 0)
    def _tables_out():
        pltpu.sync_copy(shw, wslot_hbm)
        pltpu.sync_copy(shtok, rowtok_hbm)


_RPW = NPAD // NW
_GC = (64, 64, 56)


@functools.cache
def _make_sc_gather():
    return functools.partial(
        pl.kernel, mesh=_sc_mesh(),
        out_type=jax.ShapeDtypeStruct((NPAD, D), jnp.float32),
        scratch_types=[
            pltpu.VMEM((64,), jnp.int32),
            pltpu.VMEM((64,), jnp.int32),
            pltpu.VMEM((56,), jnp.int32),
            pltpu.VMEM((64, D), jnp.float32),
            pltpu.VMEM((64, D), jnp.float32),
            pltpu.SemaphoreType.DMA,
            pltpu.SemaphoreType.DMA,
            pltpu.SemaphoreType.DMA,
            pltpu.SemaphoreType.DMA,
        ],
    )(_sc_gather_body)


def _sc_gather(rowtok, x_flat):
    return _make_sc_gather()(rowtok, x_flat)


def _sc_gather_body(rowtok_hbm, x_hbm, xs_hbm,
                    gidx0, gidx1, gidx2, buf0, buf1, gsem0, gsem1,
                    wsem0, wsem1):
    wid = lax.axis_index("s") * 2 + lax.axis_index("c")
    base = wid * _RPW
    gidx = (gidx0, gidx1, gidx2)
    bufs = (buf0, buf1, buf0)
    gsems = (gsem0, gsem1, gsem0)
    offs = (0, 64, 128)
    for i in range(3):
        pltpu.sync_copy(rowtok_hbm.at[pl.ds(base + offs[i], _GC[i])],
                        gidx[i])
    for i in range(3):
        b = bufs[i]
        dst = b if _GC[i] == 64 else b.at[pl.ds(0, _GC[i])]
        pltpu.async_copy(x_hbm.at[gidx[i]], dst, gsems[i]).wait()
        pltpu.sync_copy(dst, xs_hbm.at[pl.ds(base + offs[i], _GC[i])])


def _gmm_body(be_ref, xs_ref, ws_ref, w1_ref, b1_ref, w2_ref, b2_ref,
              w3_ref, b3_ref, o_ref):
    h = pl.program_id(1)
    x = xs_ref[...].astype(jnp.bfloat16)
    w1 = w1_ref[0].astype(jnp.bfloat16)
    w2 = w2_ref[0].astype(jnp.bfloat16)
    w3 = w3_ref[0].astype(jnp.bfloat16)
    a = jax.lax.dot_general(
        x, w1, (((1,), (1,)), ((), ())), preferred_element_type=jnp.float32
    ) + b1_ref[0]
    b = jax.lax.dot_general(
        x, w2, (((1,), (1,)), ((), ())), preferred_element_type=jnp.float32
    ) + b2_ref[0]
    ws = ws_ref[...]
    hact = ((a * jax.nn.sigmoid(a) * b) * ws).astype(jnp.bfloat16)
    y = jax.lax.dot_general(
        hact, w3, (((1,), (1,)), ((), ())), preferred_element_type=jnp.float32
    )

    @pl.when(h == 0)
    def _first():
        o_ref[...] = y + ws * b3_ref[0]

    @pl.when(h != 0)
    def _rest():
        o_ref[...] += y


def _gmm(be, xs, ws, W1, b1, W2, b2, W3, b3):
    grid_spec = pltpu.PrefetchScalarGridSpec(
        num_scalar_prefetch=1,
        grid=(NB, H // BH),
        in_specs=[
            pl.BlockSpec((BLK, D), lambda b, h, be_r: (b, 0)),
            pl.BlockSpec((BLK, 1), lambda b, h, be_r: (b, 0)),
            pl.BlockSpec((1, BH, D), lambda b, h, be_r: (be_r[b], h, 0)),
            pl.BlockSpec((1, 1, BH), lambda b, h, be_r: (be_r[b], 0, h)),
            pl.BlockSpec((1, BH, D), lambda b, h, be_r: (be_r[b], h, 0)),
            pl.BlockSpec((1, 1, BH), lambda b, h, be_r: (be_r[b], 0, h)),
            pl.BlockSpec((1, D, BH), lambda b, h, be_r: (be_r[b], 0, h)),
            pl.BlockSpec((1, 1, D), lambda b, h, be_r: (be_r[b], 0, 0)),
        ],
        out_specs=pl.BlockSpec((BLK, D), lambda b, h, be_r: (b, 0)),
    )
    return pl.pallas_call(
        _gmm_body,
        grid_spec=grid_spec,
        out_shape=jax.ShapeDtypeStruct((NPAD, D), jnp.float32),
        compiler_params=pltpu.CompilerParams(
            dimension_semantics=("arbitrary", "arbitrary"),
        ),
    )(be, xs, ws, W1, b1.reshape(E, 1, H), W2, b2.reshape(E, 1, H), W3,
      b3.reshape(E, 1, D))


_TPW = T // NW


@functools.cache
def _make_sc_combine():
    return functools.partial(
        pl.kernel, mesh=_sc_mesh(),
        out_type=(
            jax.ShapeDtypeStruct((T, D), jnp.float32),
            jax.ShapeDtypeStruct((T, D), jnp.float32),
        ),
        scratch_types=[
            pltpu.VMEM((_TPW,), jnp.int32),
            pltpu.VMEM((_TPW,), jnp.int32),
            pltpu.VMEM((_TPW, D), jnp.float32),
            pltpu.SemaphoreType.DMA,
        ],
    )(_sc_combine_body)


def _sc_combine(pos1d, ys):
    return _make_sc_combine()(pos1d, ys)


def _sc_combine_body(pos_hbm, ys_hbm, g0_hbm, g1_hbm,
                     idx0, idx1, rows_v, sem):
    wid = lax.axis_index("s") * 2 + lax.axis_index("c")
    base = wid * _TPW
    pltpu.sync_copy(pos_hbm.at[pl.ds(base, _TPW)], idx0)
    pltpu.sync_copy(pos_hbm.at[pl.ds(T + base, _TPW)], idx1)
    pltpu.async_copy(ys_hbm.at[idx0], rows_v, sem).wait()
    pltpu.sync_copy(rows_v, g0_hbm.at[pl.ds(base, _TPW)])
    pltpu.async_copy(ys_hbm.at[idx1], rows_v, sem).wait()
    pltpu.sync_copy(rows_v, g1_hbm.at[pl.ds(base, _TPW)])


def _add_body(a_ref, b_ref, o_ref):
    o_ref[...] = a_ref[...] + b_ref[...]


def _final_add(g0, g1):
    return pl.pallas_call(
        _add_body,
        grid=(4,),
        in_specs=[
            pl.BlockSpec((T // 4, D), lambda i: (i, 0)),
            pl.BlockSpec((T // 4, D), lambda i: (i, 0)),
        ],
        out_specs=pl.BlockSpec((T // 4, D), lambda i: (i, 0)),
        out_shape=jax.ShapeDtypeStruct((T, D), jnp.float32),
    )(g0, g1)


def kernel(x, Wg, W1, b1, W2, b2, W3, b3):
    B, S, _ = x.shape
    x_flat = x.reshape(T, D)

    pos, tok, prob, be, aux = _router(x_flat, Wg)
    wslot, rowtok = _sc_dispatch(
        pos.reshape(32, 128), tok.reshape(32, 128), prob.reshape(32, 128),
        (jnp.arange(NPAD, dtype=jnp.int32) * 3 + 128) % T,
        jnp.zeros((NPAD,), jnp.float32))
    xs = _sc_gather(rowtok, x_flat)
    ys = _gmm(be.reshape(NB), xs, wslot.reshape(NPAD, 1),
              W1, b1, W2, b2, W3, b3)
    g0, g1 = _sc_combine(pos.reshape(2 * T), ys)
    out = _final_add(g0, g1)
    return out.reshape(B, S, D), aux[0, 0]


# speedup vs baseline: 1.2705x; 1.0231x over previous
import functools

import jax
import jax.numpy as jnp
from jax import lax
from jax.experimental import pallas as pl
from jax.experimental.pallas import tpu as pltpu
from jax.experimental.pallas import tpu_sc as plsc

E = 8
K = 2
D = 768
H = 3072
T = 2048
BLK = 256
NB = 23
NPAD = NB * BLK
BH = 512
NW = 32
_NEG = -1e30


def _router_body(x_ref, wg_ref, pos_ref, tok_ref, prob_ref, be_ref, aux_ref,
                 m_s, mc_s):
    x = x_ref[...]
    wg = wg_ref[...]
    scores = jax.lax.dot_general(
        x, wg, (((1,), (1,)), ((), ())), preferred_element_type=jnp.float32
    )
    idx = lax.broadcasted_iota(jnp.int32, scores.shape, 1)
    m0 = jnp.max(scores, axis=1, keepdims=True)
    i0 = jnp.min(jnp.where(scores >= m0, idx, E), axis=1, keepdims=True)
    oh0 = idx == i0
    s2 = jnp.where(oh0, _NEG, scores)
    m1 = jnp.max(s2, axis=1, keepdims=True)
    i1 = jnp.min(jnp.where(s2 >= m1, idx, E), axis=1, keepdims=True)
    oh1 = idx == i1
    p0 = 1.0 / (1.0 + jnp.exp(m1 - m0))
    p1 = 1.0 - p0
    g = jnp.exp(scores - m0)
    g = g / jnp.sum(g, axis=1, keepdims=True)
    avg_g = jnp.sum(g, axis=0) * (1.0 / T)
    counts_sel = jnp.sum(jnp.where(oh0 | oh1, 1.0, 0.0), axis=0)
    aux_ref[...] = jnp.reshape(
        E * jnp.sum(counts_sel * (1.0 / T) * avg_g), (1, 1))

    m_f = jnp.concatenate(
        [jnp.where(oh0, 1.0, 0.0), jnp.where(oh1, 1.0, 0.0)], axis=0
    )
    m_s[...] = m_f
    r_io = lax.broadcasted_iota(jnp.int32, (128, 128), 0)
    c_io = lax.broadcasted_iota(jnp.int32, (128, 128), 1)
    tril = jnp.where(r_io >= c_io, 1.0, 0.0)

    def cs_body(i, base):
        blk = m_s[pl.ds(i * 128, 128), :]
        c = jax.lax.dot_general(
            tril, blk, (((1,), (0,)), ((), ())),
            preferred_element_type=jnp.float32) + base
        mc_s[pl.ds(i * 128, 128), :] = c
        return lax.slice(c, (127, 0), (128, E))

    counts = lax.fori_loop(0, (2 * T) // 128, cs_body,
                           jnp.zeros((1, E), jnp.float32))

    nb = jnp.ceil(counts * (1.0 / BLK))
    r8 = lax.broadcasted_iota(jnp.int32, (E, E), 0)
    c8 = lax.broadcasted_iota(jnp.int32, (E, E), 1)
    upper_incl = jnp.where(r8 <= c8, 1.0, 0.0)
    incl_b = jax.lax.dot_general(
        nb, upper_incl, (((1,), (0,)), ((), ())),
        preferred_element_type=jnp.float32)
    excl_b = incl_b - nb
    seg_start = excl_b * float(BLK)

    mc = mc_s[...]
    pos_f = jnp.sum(m_s[...] * (mc - 1.0 + seg_start), axis=1, keepdims=True)
    pos_ref[...] = pos_f.astype(jnp.int32)

    t_io = lax.broadcasted_iota(jnp.int32, (T, 1), 0)
    tok_all = jnp.concatenate([t_io, t_io], axis=0)
    pos_i = pos_f.astype(jnp.int32)
    init_at_pos = lax.rem(pos_i * 3 + 128, T)
    tok_ref[...] = tok_all - init_at_pos
    prob_ref[...] = jnp.concatenate([p0, p1], axis=0)

    b_io = lax.broadcasted_iota(jnp.int32, (NB, E), 0)
    excl_bi = excl_b.astype(jnp.int32)
    be = jnp.sum(jnp.where(b_io >= excl_bi, 1, 0), axis=1,
                 keepdims=True) - 1
    be_ref[...] = be


def _router(x_flat, Wg):
    return pl.pallas_call(
        _router_body,
        out_shape=(
            jax.ShapeDtypeStruct((2 * T, 1), jnp.int32),
            jax.ShapeDtypeStruct((2 * T, 1), jnp.int32),
            jax.ShapeDtypeStruct((2 * T, 1), jnp.float32),
            jax.ShapeDtypeStruct((NB, 1), jnp.int32),
            jax.ShapeDtypeStruct((1, 1), jnp.float32),
        ),
        in_specs=[
            pl.BlockSpec((T, D), lambda: (0, 0)),
            pl.BlockSpec((E, D), lambda: (0, 0)),
        ],
        out_specs=(
            pl.BlockSpec((2 * T, 1), lambda: (0, 0)),
            pl.BlockSpec((2 * T, 1), lambda: (0, 0)),
            pl.BlockSpec((2 * T, 1), lambda: (0, 0)),
            pl.BlockSpec((NB, 1), lambda: (0, 0)),
            pl.BlockSpec((1, 1), lambda: (0, 0)),
        ),
        scratch_shapes=[
            pltpu.VMEM((2 * T, E), jnp.float32),
            pltpu.VMEM((2 * T, E), jnp.float32),
        ],
    )(x_flat, Wg)


def _sc_mesh():
    return plsc.VectorSubcoreMesh(core_axis_name="c", subcore_axis_name="s")


_RPW = NPAD // NW
_GC = (64, 64, 56)


@functools.cache
def _make_sc_dispatch_gather():
    return functools.partial(
        pl.kernel, mesh=_sc_mesh(),
        out_type=(
            jax.ShapeDtypeStruct((NPAD, D), jnp.float32),
            jax.ShapeDtypeStruct((NPAD,), jnp.float32),
        ),
        scratch_types=[
            pltpu.VMEM((2, 128), jnp.int32),
            pltpu.VMEM((2, 128), jnp.int32),
            pltpu.VMEM((2, 128), jnp.float32),
            pltpu.VMEM((64,), jnp.int32),
            pltpu.VMEM((64,), jnp.int32),
            pltpu.VMEM((56,), jnp.int32),
            pltpu.VMEM((64, D), jnp.float32),
            pltpu.VMEM((64, D), jnp.float32),
            pltpu.MemorySpace.VMEM_SHARED((NPAD,), jnp.int32),
            pltpu.MemorySpace.VMEM_SHARED((NPAD,), jnp.float32),
            pltpu.SemaphoreType.DMA,
            pltpu.SemaphoreType.DMA,
        ],
    )(_sc_dispatch_gather_body)


def _sc_dispatch_gather(pos2, tok2, prob2, zi, zf, x_flat):
    return _make_sc_dispatch_gather()(pos2, tok2, prob2, zi, zf, x_flat)


def _sc_dispatch_gather_body(pos_hbm, tok_hbm, prob_hbm, zi_hbm, zf_hbm,
                             x_hbm, xs_hbm, wslot_hbm,
                             idx_v, tval_v, pval_v, gidx0, gidx1, gidx2,
                             buf0, buf1, shtok, shw, gsem0, gsem1):
    c = lax.axis_index("c")
    s = lax.axis_index("s")
    wid = s * 2 + c

    @pl.when(s == 0)
    def _zero():
        pltpu.sync_copy(zi_hbm, shtok)
        pltpu.sync_copy(zf_hbm, shw)

    plsc.subcore_barrier()
    pltpu.sync_copy(pos_hbm.at[pl.ds(s * 2, 2)], idx_v)
    pltpu.sync_copy(tok_hbm.at[pl.ds(s * 2, 2)], tval_v)
    pltpu.sync_copy(prob_hbm.at[pl.ds(s * 2, 2)], pval_v)
    for j in range(2):
        pltpu.sync_copy(tval_v.at[j], shtok.at[idx_v.at[j]], add=True)
        pltpu.sync_copy(pval_v.at[j], shw.at[idx_v.at[j]], add=True)
    plsc.subcore_barrier()

    @pl.when(wid == 0)
    def _tables_out():
        pltpu.sync_copy(shw, wslot_hbm)

    base = wid * _RPW
    gidx = (gidx0, gidx1, gidx2)
    bufs = (buf0, buf1, buf0)
    gsems = (gsem0, gsem1, gsem0)
    offs = (0, 64, 128)
    for i in range(3):
        pltpu.sync_copy(shtok.at[pl.ds(base + offs[i], _GC[i])], gidx[i])
    for i in range(3):
        b = bufs[i]
        dst = b if _GC[i] == 64 else b.at[pl.ds(0, _GC[i])]
        pltpu.async_copy(x_hbm.at[gidx[i]], dst, gsems[i]).wait()
        pltpu.sync_copy(dst, xs_hbm.at[pl.ds(base + offs[i], _GC[i])])


def _gmm_body(be_ref, xs_ref, ws_ref, w1_ref, b1_ref, w2_ref, b2_ref,
              w3_ref, b3_ref, o_ref):
    h = pl.program_id(1)
    x = xs_ref[...].astype(jnp.bfloat16)
    w1 = w1_ref[0].astype(jnp.bfloat16)
    w2 = w2_ref[0].astype(jnp.bfloat16)
    w3 = w3_ref[0].astype(jnp.bfloat16)
    a = jax.lax.dot_general(
        x, w1, (((1,), (1,)), ((), ())), preferred_element_type=jnp.float32
    ) + b1_ref[0]
    b = jax.lax.dot_general(
        x, w2, (((1,), (1,)), ((), ())), preferred_element_type=jnp.float32
    ) + b2_ref[0]
    ws = ws_ref[...]
    hact = ((a * jax.nn.sigmoid(a) * b) * ws).astype(jnp.bfloat16)
    y = jax.lax.dot_general(
        hact, w3, (((1,), (1,)), ((), ())), preferred_element_type=jnp.float32
    )

    @pl.when(h == 0)
    def _first():
        o_ref[...] = y + ws * b3_ref[0]

    @pl.when(h != 0)
    def _rest():
        o_ref[...] += y


def _gmm(be, xs, ws, W1, b1, W2, b2, W3, b3):
    grid_spec = pltpu.PrefetchScalarGridSpec(
        num_scalar_prefetch=1,
        grid=(NB, H // BH),
        in_specs=[
            pl.BlockSpec((BLK, D), lambda b, h, be_r: (b, 0)),
            pl.BlockSpec((BLK, 1), lambda b, h, be_r: (b, 0)),
            pl.BlockSpec((1, BH, D), lambda b, h, be_r: (be_r[b], h, 0)),
            pl.BlockSpec((1, 1, BH), lambda b, h, be_r: (be_r[b], 0, h)),
            pl.BlockSpec((1, BH, D), lambda b, h, be_r: (be_r[b], h, 0)),
            pl.BlockSpec((1, 1, BH), lambda b, h, be_r: (be_r[b], 0, h)),
            pl.BlockSpec((1, D, BH), lambda b, h, be_r: (be_r[b], 0, h)),
            pl.BlockSpec((1, 1, D), lambda b, h, be_r: (be_r[b], 0, 0)),
        ],
        out_specs=pl.BlockSpec((BLK, D), lambda b, h, be_r: (b, 0)),
    )
    return pl.pallas_call(
        _gmm_body,
        grid_spec=grid_spec,
        out_shape=jax.ShapeDtypeStruct((NPAD, D), jnp.float32),
        compiler_params=pltpu.CompilerParams(
            dimension_semantics=("arbitrary", "arbitrary"),
        ),
    )(be, xs, ws, W1, b1.reshape(E, 1, H), W2, b2.reshape(E, 1, H), W3,
      b3.reshape(E, 1, D))


_TPW = T // NW


@functools.cache
def _make_sc_combine():
    return functools.partial(
        pl.kernel, mesh=_sc_mesh(),
        out_type=jax.ShapeDtypeStruct((T, D), jnp.float32),
        scratch_types=[
            pltpu.VMEM((_TPW,), jnp.int32),
            pltpu.VMEM((_TPW,), jnp.int32),
            pltpu.VMEM((_TPW, D), jnp.float32),
            pltpu.VMEM((_TPW, D), jnp.float32),
            pltpu.SemaphoreType.DMA,
            pltpu.SemaphoreType.DMA,
        ],
    )(_sc_combine_body)


def _sc_combine(pos1d, ys):
    return _make_sc_combine()(pos1d, ys)


def _sc_combine_body(pos_hbm, ys_hbm, out_hbm,
                     idx0, idx1, buf0, buf1, sem0, sem1):
    wid = lax.axis_index("s") * 2 + lax.axis_index("c")
    base = wid * _TPW
    pltpu.sync_copy(pos_hbm.at[pl.ds(base, _TPW)], idx0)
    pltpu.sync_copy(pos_hbm.at[pl.ds(T + base, _TPW)], idx1)
    cp0 = pltpu.async_copy(ys_hbm.at[idx0], buf0, sem0)
    cp1 = pltpu.async_copy(ys_hbm.at[idx1], buf1, sem1)
    cp0.wait()
    cp1.wait()

    def rbody(r, cr):
        for j in range(D // 16):
            sl = pl.ds(j * 16, 16)
            buf0[r, sl] = buf0[r, sl] + buf1[r, sl]
        return cr

    lax.fori_loop(0, _TPW, rbody, 0)
    pltpu.sync_copy(buf0, out_hbm.at[pl.ds(base, _TPW)])


def kernel(x, Wg, W1, b1, W2, b2, W3, b3):
    B, S, _ = x.shape
    x_flat = x.reshape(T, D)

    pos, tok, prob, be, aux = _router(x_flat, Wg)
    xs, wslot = _sc_dispatch_gather(
        pos.reshape(32, 128), tok.reshape(32, 128), prob.reshape(32, 128),
        (jnp.arange(NPAD, dtype=jnp.int32) * 3 + 128) % T,
        jnp.zeros((NPAD,), jnp.float32), x_flat)
    ys = _gmm(be.reshape(NB), xs, wslot.reshape(NPAD, 1),
              W1, b1, W2, b2, W3, b3)
    out = _sc_combine(pos.reshape(2 * T), ys)
    return out.reshape(B, S, D), aux[0, 0]


# speedup vs baseline: 1.3434x; 1.0573x over previous
import functools

import jax
import jax.numpy as jnp
from jax import lax
from jax.experimental import pallas as pl
from jax.experimental.pallas import tpu as pltpu
from jax.experimental.pallas import tpu_sc as plsc

E = 8
K = 2
D = 768
H = 3072
T = 2048
BLK = 256
NB = 23
NPAD = NB * BLK
BH = 512
NW = 32
_NEG = -1e30


def _router_body(x_ref, wg_ref, pos_ref, tok_ref, prob_ref, be_ref, aux_ref,
                 m_s, mc_s):
    x = x_ref[...]
    wg = wg_ref[...]
    scores = jax.lax.dot_general(
        x, wg, (((1,), (1,)), ((), ())), preferred_element_type=jnp.float32
    )
    idx = lax.broadcasted_iota(jnp.int32, scores.shape, 1)
    m0 = jnp.max(scores, axis=1, keepdims=True)
    i0 = jnp.min(jnp.where(scores >= m0, idx, E), axis=1, keepdims=True)
    oh0 = idx == i0
    s2 = jnp.where(oh0, _NEG, scores)
    m1 = jnp.max(s2, axis=1, keepdims=True)
    i1 = jnp.min(jnp.where(s2 >= m1, idx, E), axis=1, keepdims=True)
    oh1 = idx == i1
    p0 = 1.0 / (1.0 + jnp.exp(m1 - m0))
    p1 = 1.0 - p0
    g = jnp.exp(scores - m0)
    g = g / jnp.sum(g, axis=1, keepdims=True)
    avg_g = jnp.sum(g, axis=0) * (1.0 / T)
    counts_sel = jnp.sum(jnp.where(oh0 | oh1, 1.0, 0.0), axis=0)
    aux_ref[...] = jnp.reshape(
        E * jnp.sum(counts_sel * (1.0 / T) * avg_g), (1, 1))

    m_f = jnp.concatenate(
        [jnp.where(oh0, 1.0, 0.0), jnp.where(oh1, 1.0, 0.0)], axis=0
    )
    m_s[...] = m_f
    r_io = lax.broadcasted_iota(jnp.int32, (128, 128), 0)
    c_io = lax.broadcasted_iota(jnp.int32, (128, 128), 1)
    tril = jnp.where(r_io >= c_io, 1.0, 0.0)

    def cs_body(i, base):
        blk = m_s[pl.ds(i * 128, 128), :]
        c = jax.lax.dot_general(
            tril, blk, (((1,), (0,)), ((), ())),
            preferred_element_type=jnp.float32) + base
        mc_s[pl.ds(i * 128, 128), :] = c
        return lax.slice(c, (127, 0), (128, E))

    counts = lax.fori_loop(0, (2 * T) // 128, cs_body,
                           jnp.zeros((1, E), jnp.float32))

    nb = jnp.ceil(counts * (1.0 / BLK))
    r8 = lax.broadcasted_iota(jnp.int32, (E, E), 0)
    c8 = lax.broadcasted_iota(jnp.int32, (E, E), 1)
    upper_incl = jnp.where(r8 <= c8, 1.0, 0.0)
    incl_b = jax.lax.dot_general(
        nb, upper_incl, (((1,), (0,)), ((), ())),
        preferred_element_type=jnp.float32)
    excl_b = incl_b - nb
    seg_start = excl_b * float(BLK)

    mc = mc_s[...]
    pos_f = jnp.sum(m_s[...] * (mc - 1.0 + seg_start), axis=1, keepdims=True)
    pos_ref[...] = pos_f.astype(jnp.int32)

    t_io = lax.broadcasted_iota(jnp.int32, (T, 1), 0)
    tok_all = jnp.concatenate([t_io, t_io], axis=0)
    pos_i = pos_f.astype(jnp.int32)
    init_at_pos = lax.rem(pos_i * 3 + 128, T)
    tok_ref[...] = tok_all - init_at_pos
    prob_ref[...] = jnp.concatenate([p0, p1], axis=0)

    b_io = lax.broadcasted_iota(jnp.int32, (NB, E), 0)
    excl_bi = excl_b.astype(jnp.int32)
    be = jnp.sum(jnp.where(b_io >= excl_bi, 1, 0), axis=1,
                 keepdims=True) - 1
    be_ref[...] = be


def _router(x_flat, Wg):
    return pl.pallas_call(
        _router_body,
        out_shape=(
            jax.ShapeDtypeStruct((2 * T, 1), jnp.int32),
            jax.ShapeDtypeStruct((2 * T, 1), jnp.int32),
            jax.ShapeDtypeStruct((2 * T, 1), jnp.float32),
            jax.ShapeDtypeStruct((NB, 1), jnp.int32),
            jax.ShapeDtypeStruct((1, 1), jnp.float32),
        ),
        in_specs=[
            pl.BlockSpec((T, D), lambda: (0, 0)),
            pl.BlockSpec((E, D), lambda: (0, 0)),
        ],
        out_specs=(
            pl.BlockSpec((2 * T, 1), lambda: (0, 0)),
            pl.BlockSpec((2 * T, 1), lambda: (0, 0)),
            pl.BlockSpec((2 * T, 1), lambda: (0, 0)),
            pl.BlockSpec((NB, 1), lambda: (0, 0)),
            pl.BlockSpec((1, 1), lambda: (0, 0)),
        ),
        scratch_shapes=[
            pltpu.VMEM((2 * T, E), jnp.float32),
            pltpu.VMEM((2 * T, E), jnp.float32),
        ],
    )(x_flat, Wg)


def _sc_mesh():
    return plsc.VectorSubcoreMesh(core_axis_name="c", subcore_axis_name="s")


_RPW = NPAD // NW
_GC = (64, 64, 56)


@functools.cache
def _make_sc_dispatch_gather():
    return functools.partial(
        pl.kernel, mesh=_sc_mesh(),
        out_type=(
            jax.ShapeDtypeStruct((NPAD, D), jnp.float32),
            jax.ShapeDtypeStruct((NPAD,), jnp.float32),
        ),
        scratch_types=[
            pltpu.VMEM((2, 128), jnp.int32),
            pltpu.VMEM((2, 128), jnp.int32),
            pltpu.VMEM((2, 128), jnp.float32),
            pltpu.VMEM((64,), jnp.int32),
            pltpu.VMEM((64,), jnp.int32),
            pltpu.VMEM((56,), jnp.int32),
            pltpu.VMEM((64, D), jnp.float32),
            pltpu.VMEM((64, D), jnp.float32),
            pltpu.MemorySpace.VMEM_SHARED((NPAD,), jnp.int32),
            pltpu.MemorySpace.VMEM_SHARED((NPAD,), jnp.float32),
            pltpu.SemaphoreType.DMA,
            pltpu.SemaphoreType.DMA,
        ],
    )(_sc_dispatch_gather_body)


def _sc_dispatch_gather(pos2, tok2, prob2, zi, zf, x_flat):
    return _make_sc_dispatch_gather()(pos2, tok2, prob2, zi, zf, x_flat)


def _sc_dispatch_gather_body(pos_hbm, tok_hbm, prob_hbm, zi_hbm, zf_hbm,
                             x_hbm, xs_hbm, wslot_hbm,
                             idx_v, tval_v, pval_v, gidx0, gidx1, gidx2,
                             buf0, buf1, shtok, shw, gsem0, gsem1):
    c = lax.axis_index("c")
    s = lax.axis_index("s")
    wid = s * 2 + c

    @pl.when(s == 0)
    def _zero():
        pltpu.sync_copy(zi_hbm, shtok)
        pltpu.sync_copy(zf_hbm, shw)

    plsc.subcore_barrier()
    pltpu.sync_copy(pos_hbm.at[pl.ds(s * 2, 2)], idx_v)
    pltpu.sync_copy(tok_hbm.at[pl.ds(s * 2, 2)], tval_v)
    pltpu.sync_copy(prob_hbm.at[pl.ds(s * 2, 2)], pval_v)
    for j in range(2):
        pltpu.sync_copy(tval_v.at[j], shtok.at[idx_v.at[j]], add=True)
        pltpu.sync_copy(pval_v.at[j], shw.at[idx_v.at[j]], add=True)
    plsc.subcore_barrier()

    @pl.when(wid == 0)
    def _tables_out():
        pltpu.sync_copy(shw, wslot_hbm)

    base = wid * _RPW
    gidx = (gidx0, gidx1, gidx2)
    bufs = (buf0, buf1, buf0)
    gsems = (gsem0, gsem1, gsem0)
    offs = (0, 64, 128)
    for i in range(3):
        pltpu.sync_copy(shtok.at[pl.ds(base + offs[i], _GC[i])], gidx[i])
    for i in range(3):
        b = bufs[i]
        dst = b if _GC[i] == 64 else b.at[pl.ds(0, _GC[i])]
        pltpu.async_copy(x_hbm.at[gidx[i]], dst, gsems[i]).wait()
        pltpu.sync_copy(dst, xs_hbm.at[pl.ds(base + offs[i], _GC[i])])


def _gmm_body(be_ref, xs_ref, ws_ref, w1_ref, b1_ref, w2_ref, b2_ref,
              w3_ref, b3_ref, o_ref):
    h = pl.program_id(0)
    b = pl.program_id(1)
    x = xs_ref[...].astype(jnp.bfloat16)
    w1 = w1_ref[0].astype(jnp.bfloat16)
    w2 = w2_ref[0].astype(jnp.bfloat16)
    w3 = w3_ref[0].astype(jnp.bfloat16)
    a = jax.lax.dot_general(
        x, w1, (((1,), (1,)), ((), ())), preferred_element_type=jnp.float32
    ) + b1_ref[0]
    bb = jax.lax.dot_general(
        x, w2, (((1,), (1,)), ((), ())), preferred_element_type=jnp.float32
    ) + b2_ref[0]
    ws = ws_ref[...]
    hact = ((a * jax.nn.sigmoid(a) * bb) * ws).astype(jnp.bfloat16)
    y = jax.lax.dot_general(
        hact, w3, (((1,), (1,)), ((), ())), preferred_element_type=jnp.float32
    )
    row = b * BLK

    @pl.when(h == 0)
    def _first():
        o_ref[pl.ds(row, BLK), :] = y + ws * b3_ref[0]

    @pl.when(h != 0)
    def _rest():
        o_ref[pl.ds(row, BLK), :] += y


def _gmm(be, xs, ws, W1, b1, W2, b2, W3, b3):
    grid_spec = pltpu.PrefetchScalarGridSpec(
        num_scalar_prefetch=1,
        grid=(H // BH, NB),
        in_specs=[
            pl.BlockSpec((BLK, D), lambda h, b, be_r: (b, 0)),
            pl.BlockSpec((BLK, 1), lambda h, b, be_r: (b, 0)),
            pl.BlockSpec((1, BH, D), lambda h, b, be_r: (be_r[b], h, 0)),
            pl.BlockSpec((1, 1, BH), lambda h, b, be_r: (be_r[b], 0, h)),
            pl.BlockSpec((1, BH, D), lambda h, b, be_r: (be_r[b], h, 0)),
            pl.BlockSpec((1, 1, BH), lambda h, b, be_r: (be_r[b], 0, h)),
            pl.BlockSpec((1, D, BH), lambda h, b, be_r: (be_r[b], 0, h)),
            pl.BlockSpec((1, 1, D), lambda h, b, be_r: (be_r[b], 0, 0)),
        ],
        out_specs=pl.BlockSpec((NPAD, D), lambda h, b, be_r: (0, 0)),
    )
    return pl.pallas_call(
        _gmm_body,
        grid_spec=grid_spec,
        out_shape=jax.ShapeDtypeStruct((NPAD, D), jnp.float32),
        compiler_params=pltpu.CompilerParams(
            dimension_semantics=("arbitrary", "arbitrary"),
        ),
    )(be, xs, ws, W1, b1.reshape(E, 1, H), W2, b2.reshape(E, 1, H), W3,
      b3.reshape(E, 1, D))


_TPW = T // NW


@functools.cache
def _make_sc_combine():
    return functools.partial(
        pl.kernel, mesh=_sc_mesh(),
        out_type=jax.ShapeDtypeStruct((T, D), jnp.float32),
        scratch_types=[
            pltpu.VMEM((_TPW,), jnp.int32),
            pltpu.VMEM((_TPW,), jnp.int32),
            pltpu.VMEM((_TPW, D), jnp.float32),
            pltpu.VMEM((_TPW, D), jnp.float32),
            pltpu.SemaphoreType.DMA,
            pltpu.SemaphoreType.DMA,
        ],
    )(_sc_combine_body)


def _sc_combine(pos1d, ys):
    return _make_sc_combine()(pos1d, ys)


def _sc_combine_body(pos_hbm, ys_hbm, out_hbm,
                     idx0, idx1, buf0, buf1, sem0, sem1):
    wid = lax.axis_index("s") * 2 + lax.axis_index("c")
    base = wid * _TPW
    pltpu.sync_copy(pos_hbm.at[pl.ds(base, _TPW)], idx0)
    pltpu.sync_copy(pos_hbm.at[pl.ds(T + base, _TPW)], idx1)
    cp0 = pltpu.async_copy(ys_hbm.at[idx0], buf0, sem0)
    cp1 = pltpu.async_copy(ys_hbm.at[idx1], buf1, sem1)
    cp0.wait()
    cp1.wait()

    def rbody(r, cr):
        for j in range(D // 16):
            sl = pl.ds(j * 16, 16)
            buf0[r, sl] = buf0[r, sl] + buf1[r, sl]
        return cr

    lax.fori_loop(0, _TPW, rbody, 0)
    pltpu.sync_copy(buf0, out_hbm.at[pl.ds(base, _TPW)])


def kernel(x, Wg, W1, b1, W2, b2, W3, b3):
    B, S, _ = x.shape
    x_flat = x.reshape(T, D)

    pos, tok, prob, be, aux = _router(x_flat, Wg)
    xs, wslot = _sc_dispatch_gather(
        pos.reshape(32, 128), tok.reshape(32, 128), prob.reshape(32, 128),
        (jnp.arange(NPAD, dtype=jnp.int32) * 3 + 128) % T,
        jnp.zeros((NPAD,), jnp.float32), x_flat)
    ys = _gmm(be.reshape(NB), xs, wslot.reshape(NPAD, 1),
              W1, b1, W2, b2, W3, b3)
    out = _sc_combine(pos.reshape(2 * T), ys)
    return out.reshape(B, S, D), aux[0, 0]


# speedup vs baseline: 1.5742x; 1.1718x over previous
import functools

import jax
import jax.numpy as jnp
from jax import lax
from jax.experimental import pallas as pl
from jax.experimental.pallas import tpu as pltpu
from jax.experimental.pallas import tpu_sc as plsc

E = 8
K = 2
D = 768
H = 3072
T = 2048
BLK = 256
NB = 23
NPAD = NB * BLK
BH = 768
NW = 32
_NEG = -1e30


def _router_body(x_ref, wg_ref, pos_ref, tok_ref, prob_ref, be_ref, aux_ref,
                 m_s, mc_s):
    x = x_ref[...]
    wg = wg_ref[...]
    scores = jax.lax.dot_general(
        x, wg, (((1,), (1,)), ((), ())), preferred_element_type=jnp.float32
    )
    idx = lax.broadcasted_iota(jnp.int32, scores.shape, 1)
    m0 = jnp.max(scores, axis=1, keepdims=True)
    i0 = jnp.min(jnp.where(scores >= m0, idx, E), axis=1, keepdims=True)
    oh0 = idx == i0
    s2 = jnp.where(oh0, _NEG, scores)
    m1 = jnp.max(s2, axis=1, keepdims=True)
    i1 = jnp.min(jnp.where(s2 >= m1, idx, E), axis=1, keepdims=True)
    oh1 = idx == i1
    p0 = 1.0 / (1.0 + jnp.exp(m1 - m0))
    p1 = 1.0 - p0
    g = jnp.exp(scores - m0)
    g = g / jnp.sum(g, axis=1, keepdims=True)
    avg_g = jnp.sum(g, axis=0) * (1.0 / T)
    counts_sel = jnp.sum(jnp.where(oh0 | oh1, 1.0, 0.0), axis=0)
    aux_ref[...] = jnp.reshape(
        E * jnp.sum(counts_sel * (1.0 / T) * avg_g), (1, 1))

    m_f = jnp.concatenate(
        [jnp.where(oh0, 1.0, 0.0), jnp.where(oh1, 1.0, 0.0)], axis=0
    )
    m_s[...] = m_f
    r_io = lax.broadcasted_iota(jnp.int32, (128, 128), 0)
    c_io = lax.broadcasted_iota(jnp.int32, (128, 128), 1)
    tril = jnp.where(r_io >= c_io, 1.0, 0.0)

    def cs_body(i, base):
        blk = m_s[pl.ds(i * 128, 128), :]
        c = jax.lax.dot_general(
            tril, blk, (((1,), (0,)), ((), ())),
            preferred_element_type=jnp.float32) + base
        mc_s[pl.ds(i * 128, 128), :] = c
        return lax.slice(c, (127, 0), (128, E))

    counts = lax.fori_loop(0, (2 * T) // 128, cs_body,
                           jnp.zeros((1, E), jnp.float32))

    nb = jnp.ceil(counts * (1.0 / BLK))
    r8 = lax.broadcasted_iota(jnp.int32, (E, E), 0)
    c8 = lax.broadcasted_iota(jnp.int32, (E, E), 1)
    upper_incl = jnp.where(r8 <= c8, 1.0, 0.0)
    incl_b = jax.lax.dot_general(
        nb, upper_incl, (((1,), (0,)), ((), ())),
        preferred_element_type=jnp.float32)
    excl_b = incl_b - nb
    seg_start = excl_b * float(BLK)

    mc = mc_s[...]
    pos_f = jnp.sum(m_s[...] * (mc - 1.0 + seg_start), axis=1, keepdims=True)
    pos_ref[...] = pos_f.astype(jnp.int32)

    t_io = lax.broadcasted_iota(jnp.int32, (T, 1), 0)
    tok_all = jnp.concatenate([t_io, t_io], axis=0)
    pos_i = pos_f.astype(jnp.int32)
    init_at_pos = lax.rem(pos_i * 3 + 128, T)
    tok_ref[...] = tok_all - init_at_pos
    prob_ref[...] = jnp.concatenate([p0, p1], axis=0)

    b_io = lax.broadcasted_iota(jnp.int32, (NB + 1, E), 0)
    excl_bi = excl_b.astype(jnp.int32)
    be = jnp.sum(jnp.where(b_io >= excl_bi, 1, 0), axis=1,
                 keepdims=True) - 1
    used = jnp.sum(nb, axis=1, keepdims=True).astype(jnp.int32)
    row_i = lax.broadcasted_iota(jnp.int32, (NB + 1, 1), 0)
    be_ref[...] = jnp.where(row_i == NB, used, be)


def _router(x_flat, Wg):
    return pl.pallas_call(
        _router_body,
        out_shape=(
            jax.ShapeDtypeStruct((2 * T, 1), jnp.int32),
            jax.ShapeDtypeStruct((2 * T, 1), jnp.int32),
            jax.ShapeDtypeStruct((2 * T, 1), jnp.float32),
            jax.ShapeDtypeStruct((NB + 1, 1), jnp.int32),
            jax.ShapeDtypeStruct((1, 1), jnp.float32),
        ),
        in_specs=[
            pl.BlockSpec((T, D), lambda: (0, 0)),
            pl.BlockSpec((E, D), lambda: (0, 0)),
        ],
        out_specs=(
            pl.BlockSpec((2 * T, 1), lambda: (0, 0)),
            pl.BlockSpec((2 * T, 1), lambda: (0, 0)),
            pl.BlockSpec((2 * T, 1), lambda: (0, 0)),
            pl.BlockSpec((NB + 1, 1), lambda: (0, 0)),
            pl.BlockSpec((1, 1), lambda: (0, 0)),
        ),
        scratch_shapes=[
            pltpu.VMEM((2 * T, E), jnp.float32),
            pltpu.VMEM((2 * T, E), jnp.float32),
        ],
    )(x_flat, Wg)


def _sc_mesh():
    return plsc.VectorSubcoreMesh(core_axis_name="c", subcore_axis_name="s")


_RPW = NPAD // NW
_GC = (64, 64, 56)


@functools.cache
def _make_sc_dispatch_gather():
    return functools.partial(
        pl.kernel, mesh=_sc_mesh(),
        out_type=(
            jax.ShapeDtypeStruct((NPAD, D), jnp.float32),
            jax.ShapeDtypeStruct((NPAD,), jnp.float32),
        ),
        scratch_types=[
            pltpu.VMEM((2, 128), jnp.int32),
            pltpu.VMEM((2, 128), jnp.int32),
            pltpu.VMEM((2, 128), jnp.float32),
            pltpu.VMEM((64,), jnp.int32),
            pltpu.VMEM((64,), jnp.int32),
            pltpu.VMEM((56,), jnp.int32),
            pltpu.VMEM((64, D), jnp.float32),
            pltpu.VMEM((64, D), jnp.float32),
            pltpu.MemorySpace.VMEM_SHARED((NPAD,), jnp.int32),
            pltpu.MemorySpace.VMEM_SHARED((NPAD,), jnp.float32),
            pltpu.SemaphoreType.DMA,
            pltpu.SemaphoreType.DMA,
        ],
    )(_sc_dispatch_gather_body)


def _sc_dispatch_gather(pos2, tok2, prob2, zi, zf, x_flat):
    return _make_sc_dispatch_gather()(pos2, tok2, prob2, zi, zf, x_flat)


def _sc_dispatch_gather_body(pos_hbm, tok_hbm, prob_hbm, zi_hbm, zf_hbm,
                             x_hbm, xs_hbm, wslot_hbm,
                             idx_v, tval_v, pval_v, gidx0, gidx1, gidx2,
                             buf0, buf1, shtok, shw, gsem0, gsem1):
    c = lax.axis_index("c")
    s = lax.axis_index("s")
    wid = s * 2 + c

    @pl.when(s == 0)
    def _zero():
        pltpu.sync_copy(zi_hbm, shtok)
        pltpu.sync_copy(zf_hbm, shw)

    plsc.subcore_barrier()
    pltpu.sync_copy(pos_hbm.at[pl.ds(s * 2, 2)], idx_v)
    pltpu.sync_copy(tok_hbm.at[pl.ds(s * 2, 2)], tval_v)
    pltpu.sync_copy(prob_hbm.at[pl.ds(s * 2, 2)], pval_v)
    for j in range(2):
        pltpu.sync_copy(tval_v.at[j], shtok.at[idx_v.at[j]], add=True)
        pltpu.sync_copy(pval_v.at[j], shw.at[idx_v.at[j]], add=True)
    plsc.subcore_barrier()

    @pl.when(wid == 0)
    def _tables_out():
        pltpu.sync_copy(shw, wslot_hbm)

    base = wid * _RPW
    gidx = (gidx0, gidx1, gidx2)
    bufs = (buf0, buf1, buf0)
    gsems = (gsem0, gsem1, gsem0)
    offs = (0, 64, 128)
    for i in range(3):
        pltpu.sync_copy(shtok.at[pl.ds(base + offs[i], _GC[i])], gidx[i])
    for i in range(3):
        b = bufs[i]
        dst = b if _GC[i] == 64 else b.at[pl.ds(0, _GC[i])]
        pltpu.async_copy(x_hbm.at[gidx[i]], dst, gsems[i]).wait()
        pltpu.sync_copy(dst, xs_hbm.at[pl.ds(base + offs[i], _GC[i])])


def _gmm_body(be_ref, xs_ref, ws_ref, w1_ref, b1_ref, w2_ref, b2_ref,
              w3_ref, b3_ref, o_ref):
    h = pl.program_id(0)
    b = pl.program_id(1)
    used = be_ref[NB]

    @pl.when(b < used)
    def _compute():
        x = xs_ref[...].astype(jnp.bfloat16)
        w1 = w1_ref[0].astype(jnp.bfloat16)
        w2 = w2_ref[0].astype(jnp.bfloat16)
        w3 = w3_ref[0].astype(jnp.bfloat16)
        a = jax.lax.dot_general(
            x, w1, (((1,), (1,)), ((), ())),
            preferred_element_type=jnp.float32) + b1_ref[0]
        bb = jax.lax.dot_general(
            x, w2, (((1,), (1,)), ((), ())),
            preferred_element_type=jnp.float32) + b2_ref[0]
        ws = ws_ref[...]
        hact = ((a * jax.nn.sigmoid(a) * bb) * ws).astype(jnp.bfloat16)
        y = jax.lax.dot_general(
            hact, w3, (((1,), (1,)), ((), ())),
            preferred_element_type=jnp.float32)
        row = b * BLK

        @pl.when(h == 0)
        def _first():
            o_ref[pl.ds(row, BLK), :] = y + ws * b3_ref[0]

        @pl.when(h != 0)
        def _rest():
            o_ref[pl.ds(row, BLK), :] += y


def _gmm(be, xs, ws, W1, b1, W2, b2, W3, b3):
    grid_spec = pltpu.PrefetchScalarGridSpec(
        num_scalar_prefetch=1,
        grid=(H // BH, NB),
        in_specs=[
            pl.BlockSpec((BLK, D), lambda h, b, be_r: (b, 0)),
            pl.BlockSpec((BLK, 1), lambda h, b, be_r: (b, 0)),
            pl.BlockSpec((1, BH, D), lambda h, b, be_r: (be_r[b], h, 0)),
            pl.BlockSpec((1, 1, BH), lambda h, b, be_r: (be_r[b], 0, h)),
            pl.BlockSpec((1, BH, D), lambda h, b, be_r: (be_r[b], h, 0)),
            pl.BlockSpec((1, 1, BH), lambda h, b, be_r: (be_r[b], 0, h)),
            pl.BlockSpec((1, D, BH), lambda h, b, be_r: (be_r[b], 0, h)),
            pl.BlockSpec((1, 1, D), lambda h, b, be_r: (be_r[b], 0, 0)),
        ],
        out_specs=pl.BlockSpec((NPAD, D), lambda h, b, be_r: (0, 0)),
    )
    return pl.pallas_call(
        _gmm_body,
        grid_spec=grid_spec,
        out_shape=jax.ShapeDtypeStruct((NPAD, D), jnp.float32),
        compiler_params=pltpu.CompilerParams(
            dimension_semantics=("arbitrary", "arbitrary"),
        ),
    )(be, xs, ws, W1, b1.reshape(E, 1, H), W2, b2.reshape(E, 1, H), W3,
      b3.reshape(E, 1, D))


_TPW = T // NW


@functools.cache
def _make_sc_combine():
    return functools.partial(
        pl.kernel, mesh=_sc_mesh(),
        out_type=jax.ShapeDtypeStruct((T, D), jnp.float32),
        scratch_types=[
            pltpu.VMEM((_TPW,), jnp.int32),
            pltpu.VMEM((_TPW,), jnp.int32),
            pltpu.VMEM((_TPW, D), jnp.float32),
            pltpu.VMEM((_TPW, D), jnp.float32),
            pltpu.SemaphoreType.DMA,
            pltpu.SemaphoreType.DMA,
        ],
    )(_sc_combine_body)


def _sc_combine(pos1d, ys):
    return _make_sc_combine()(pos1d, ys)


def _sc_combine_body(pos_hbm, ys_hbm, out_hbm,
                     idx0, idx1, buf0, buf1, sem0, sem1):
    wid = lax.axis_index("s") * 2 + lax.axis_index("c")
    base = wid * _TPW
    pltpu.sync_copy(pos_hbm.at[pl.ds(base, _TPW)], idx0)
    pltpu.sync_copy(pos_hbm.at[pl.ds(T + base, _TPW)], idx1)
    cp0 = pltpu.async_copy(ys_hbm.at[idx0], buf0, sem0)
    cp1 = pltpu.async_copy(ys_hbm.at[idx1], buf1, sem1)
    cp0.wait()
    cp1.wait()

    def rbody(r, cr):
        for j in range(D // 16):
            sl = pl.ds(j * 16, 16)
            buf0[r, sl] = buf0[r, sl] + buf1[r, sl]
        return cr

    lax.fori_loop(0, _TPW, rbody, 0)
    pltpu.sync_copy(buf0, out_hbm.at[pl.ds(base, _TPW)])


def kernel(x, Wg, W1, b1, W2, b2, W3, b3):
    B, S, _ = x.shape
    x_flat = x.reshape(T, D)

    pos, tok, prob, be, aux = _router(x_flat, Wg)
    xs, wslot = _sc_dispatch_gather(
        pos.reshape(32, 128), tok.reshape(32, 128), prob.reshape(32, 128),
        (jnp.arange(NPAD, dtype=jnp.int32) * 3 + 128) % T,
        jnp.zeros((NPAD,), jnp.float32), x_flat)
    ys = _gmm(be.reshape(NB + 1), xs, wslot.reshape(NPAD, 1),
              W1, b1, W2, b2, W3, b3)
    out = _sc_combine(pos.reshape(2 * T), ys)
    return out.reshape(B, S, D), aux[0, 0]


# speedup vs baseline: 1.6306x; 1.0358x over previous
import functools

import jax
import jax.numpy as jnp
from jax import lax
from jax.experimental import pallas as pl
from jax.experimental.pallas import tpu as pltpu
from jax.experimental.pallas import tpu_sc as plsc

E = 8
K = 2
D = 768
H = 3072
T = 2048
BLK = 256
NB = 23
NPAD = NB * BLK
BH = 768
NW = 32
_NEG = -1e30


def _router_body(x_ref, wg_ref, pos_ref, tok_ref, prob_ref, be_ref, aux_ref,
                 m_s, mc_s):
    x = x_ref[...]
    wg = wg_ref[...]
    scores = jax.lax.dot_general(
        x, wg, (((1,), (1,)), ((), ())), preferred_element_type=jnp.float32
    )
    idx = lax.broadcasted_iota(jnp.int32, scores.shape, 1)
    m0 = jnp.max(scores, axis=1, keepdims=True)
    i0 = jnp.min(jnp.where(scores >= m0, idx, E), axis=1, keepdims=True)
    oh0 = idx == i0
    s2 = jnp.where(oh0, _NEG, scores)
    m1 = jnp.max(s2, axis=1, keepdims=True)
    i1 = jnp.min(jnp.where(s2 >= m1, idx, E), axis=1, keepdims=True)
    oh1 = idx == i1
    p0 = 1.0 / (1.0 + jnp.exp(m1 - m0))
    p1 = 1.0 - p0
    g = jnp.exp(scores - m0)
    g = g / jnp.sum(g, axis=1, keepdims=True)
    avg_g = jnp.sum(g, axis=0) * (1.0 / T)
    counts_sel = jnp.sum(jnp.where(oh0 | oh1, 1.0, 0.0), axis=0)
    aux_ref[...] = jnp.reshape(
        E * jnp.sum(counts_sel * (1.0 / T) * avg_g), (1, 1))

    m_f = jnp.concatenate(
        [jnp.where(oh0, 1.0, 0.0), jnp.where(oh1, 1.0, 0.0)], axis=0
    )
    m_s[...] = m_f
    r_io = lax.broadcasted_iota(jnp.int32, (128, 128), 0)
    c_io = lax.broadcasted_iota(jnp.int32, (128, 128), 1)
    tril = jnp.where(r_io >= c_io, 1.0, 0.0)

    def cs_body(i, base):
        blk = m_s[pl.ds(i * 128, 128), :]
        c = jax.lax.dot_general(
            tril, blk, (((1,), (0,)), ((), ())),
            preferred_element_type=jnp.float32) + base
        mc_s[pl.ds(i * 128, 128), :] = c
        return lax.slice(c, (127, 0), (128, E))

    counts = lax.fori_loop(0, (2 * T) // 128, cs_body,
                           jnp.zeros((1, E), jnp.float32))

    nb = jnp.ceil(counts * (1.0 / BLK))
    r8 = lax.broadcasted_iota(jnp.int32, (E, E), 0)
    c8 = lax.broadcasted_iota(jnp.int32, (E, E), 1)
    upper_incl = jnp.where(r8 <= c8, 1.0, 0.0)
    incl_b = jax.lax.dot_general(
        nb, upper_incl, (((1,), (0,)), ((), ())),
        preferred_element_type=jnp.float32)
    excl_b = incl_b - nb
    seg_start = excl_b * float(BLK)

    mc = mc_s[...]
    pos_f = jnp.sum(m_s[...] * (mc - 1.0 + seg_start), axis=1, keepdims=True)
    pos_ref[...] = jnp.reshape(pos_f.astype(jnp.int32), (32, 128))

    t_io = lax.broadcasted_iota(jnp.int32, (T, 1), 0)
    tok_all = jnp.concatenate([t_io, t_io], axis=0)
    pos_i = pos_f.astype(jnp.int32)
    init_at_pos = lax.rem(pos_i * 3 + 128, T)
    tok_ref[...] = jnp.reshape(tok_all - init_at_pos, (32, 128))
    prob_ref[...] = jnp.reshape(jnp.concatenate([p0, p1], axis=0),
                                (32, 128))

    b_io = lax.broadcasted_iota(jnp.int32, (NB + 1, E), 0)
    excl_bi = excl_b.astype(jnp.int32)
    be = jnp.sum(jnp.where(b_io >= excl_bi, 1, 0), axis=1,
                 keepdims=True) - 1
    used = jnp.sum(nb, axis=1, keepdims=True).astype(jnp.int32)
    row_i = lax.broadcasted_iota(jnp.int32, (NB + 1, 1), 0)
    be_ref[...] = jnp.where(row_i == NB, used, be)


def _router(x_flat, Wg):
    return pl.pallas_call(
        _router_body,
        out_shape=(
            jax.ShapeDtypeStruct((32, 128), jnp.int32),
            jax.ShapeDtypeStruct((32, 128), jnp.int32),
            jax.ShapeDtypeStruct((32, 128), jnp.float32),
            jax.ShapeDtypeStruct((NB + 1, 1), jnp.int32),
            jax.ShapeDtypeStruct((1, 1), jnp.float32),
        ),
        in_specs=[
            pl.BlockSpec((T, D), lambda: (0, 0)),
            pl.BlockSpec((E, D), lambda: (0, 0)),
        ],
        out_specs=(
            pl.BlockSpec((32, 128), lambda: (0, 0)),
            pl.BlockSpec((32, 128), lambda: (0, 0)),
            pl.BlockSpec((32, 128), lambda: (0, 0)),
            pl.BlockSpec((NB + 1, 1), lambda: (0, 0)),
            pl.BlockSpec((1, 1), lambda: (0, 0)),
        ),
        scratch_shapes=[
            pltpu.VMEM((2 * T, E), jnp.float32),
            pltpu.VMEM((2 * T, E), jnp.float32),
        ],
    )(x_flat, Wg)


def _sc_mesh():
    return plsc.VectorSubcoreMesh(core_axis_name="c", subcore_axis_name="s")


_RPW = NPAD // NW
_GC = (64, 64, 56)


@functools.cache
def _make_sc_dispatch_gather():
    return functools.partial(
        pl.kernel, mesh=_sc_mesh(),
        out_type=(
            jax.ShapeDtypeStruct((NPAD, D), jnp.float32),
            jax.ShapeDtypeStruct((NPAD,), jnp.float32),
        ),
        scratch_types=[
            pltpu.VMEM((2, 128), jnp.int32),
            pltpu.VMEM((2, 128), jnp.int32),
            pltpu.VMEM((2, 128), jnp.float32),
            pltpu.VMEM((64,), jnp.int32),
            pltpu.VMEM((64,), jnp.int32),
            pltpu.VMEM((56,), jnp.int32),
            pltpu.VMEM((64, D), jnp.float32),
            pltpu.VMEM((64, D), jnp.float32),
            pltpu.MemorySpace.VMEM_SHARED((NPAD,), jnp.int32),
            pltpu.MemorySpace.VMEM_SHARED((NPAD,), jnp.float32),
            pltpu.SemaphoreType.DMA,
            pltpu.SemaphoreType.DMA,
        ],
    )(_sc_dispatch_gather_body)


def _sc_dispatch_gather(pos2, tok2, prob2, zi, zf, x_flat):
    return _make_sc_dispatch_gather()(pos2, tok2, prob2, zi, zf, x_flat)


def _sc_dispatch_gather_body(pos_hbm, tok_hbm, prob_hbm, zi_hbm, zf_hbm,
                             x_hbm, xs_hbm, wslot_hbm,
                             idx_v, tval_v, pval_v, gidx0, gidx1, gidx2,
                             buf0, buf1, shtok, shw, gsem0, gsem1):
    c = lax.axis_index("c")
    s = lax.axis_index("s")
    wid = s * 2 + c

    @pl.when(s == 0)
    def _zero():
        pltpu.sync_copy(zi_hbm, shtok)
        pltpu.sync_copy(zf_hbm, shw)

    plsc.subcore_barrier()
    pltpu.sync_copy(pos_hbm.at[pl.ds(s * 2, 2)], idx_v)
    pltpu.sync_copy(tok_hbm.at[pl.ds(s * 2, 2)], tval_v)
    pltpu.sync_copy(prob_hbm.at[pl.ds(s * 2, 2)], pval_v)
    for j in range(2):
        pltpu.sync_copy(tval_v.at[j], shtok.at[idx_v.at[j]], add=True)
        pltpu.sync_copy(pval_v.at[j], shw.at[idx_v.at[j]], add=True)
    plsc.subcore_barrier()

    @pl.when(wid == 0)
    def _tables_out():
        pltpu.sync_copy(shw, wslot_hbm)

    base = wid * _RPW
    gidx = (gidx0, gidx1, gidx2)
    bufs = (buf0, buf1, buf0)
    gsems = (gsem0, gsem1, gsem0)
    offs = (0, 64, 128)
    for i in range(3):
        pltpu.sync_copy(shtok.at[pl.ds(base + offs[i], _GC[i])], gidx[i])
    for i in range(3):
        b = bufs[i]
        dst = b if _GC[i] == 64 else b.at[pl.ds(0, _GC[i])]
        pltpu.async_copy(x_hbm.at[gidx[i]], dst, gsems[i]).wait()
        pltpu.sync_copy(dst, xs_hbm.at[pl.ds(base + offs[i], _GC[i])])


def _gmm_body(be_ref, xs_ref, ws_ref, w1_ref, b1_ref, w2_ref, b2_ref,
              w3_ref, b3_ref, o_ref):
    h = pl.program_id(0)
    b = pl.program_id(1)
    used = be_ref[NB]

    @pl.when(b < used)
    def _compute():
        x = xs_ref[...].astype(jnp.bfloat16)
        w1 = w1_ref[0].astype(jnp.bfloat16)
        w2 = w2_ref[0].astype(jnp.bfloat16)
        w3 = w3_ref[0].astype(jnp.bfloat16)
        a = jax.lax.dot_general(
            x, w1, (((1,), (1,)), ((), ())),
            preferred_element_type=jnp.float32) + b1_ref[0]
        bb = jax.lax.dot_general(
            x, w2, (((1,), (1,)), ((), ())),
            preferred_element_type=jnp.float32) + b2_ref[0]
        ws = ws_ref[...]
        hact = ((a * jax.nn.sigmoid(a) * bb) * ws).astype(jnp.bfloat16)
        y = jax.lax.dot_general(
            hact, w3, (((1,), (1,)), ((), ())),
            preferred_element_type=jnp.float32)
        row = b * BLK

        @pl.when(h == 0)
        def _first():
            o_ref[pl.ds(row, BLK), :] = y + ws * b3_ref[0]

        @pl.when(h != 0)
        def _rest():
            o_ref[pl.ds(row, BLK), :] += y


def _gmm(be, xs, ws, W1, b1, W2, b2, W3, b3):
    grid_spec = pltpu.PrefetchScalarGridSpec(
        num_scalar_prefetch=1,
        grid=(H // BH, NB),
        in_specs=[
            pl.BlockSpec((BLK, D), lambda h, b, be_r: (b, 0)),
            pl.BlockSpec((BLK, 1), lambda h, b, be_r: (b, 0)),
            pl.BlockSpec((1, BH, D), lambda h, b, be_r: (be_r[b], h, 0)),
            pl.BlockSpec((1, 1, BH), lambda h, b, be_r: (be_r[b], 0, h)),
            pl.BlockSpec((1, BH, D), lambda h, b, be_r: (be_r[b], h, 0)),
            pl.BlockSpec((1, 1, BH), lambda h, b, be_r: (be_r[b], 0, h)),
            pl.BlockSpec((1, D, BH), lambda h, b, be_r: (be_r[b], 0, h)),
            pl.BlockSpec((1, 1, D), lambda h, b, be_r: (be_r[b], 0, 0)),
        ],
        out_specs=pl.BlockSpec((NPAD, D), lambda h, b, be_r: (0, 0)),
    )
    return pl.pallas_call(
        _gmm_body,
        grid_spec=grid_spec,
        out_shape=jax.ShapeDtypeStruct((NPAD, D), jnp.float32),
        compiler_params=pltpu.CompilerParams(
            dimension_semantics=("arbitrary", "arbitrary"),
        ),
    )(be, xs, ws, W1, b1.reshape(E, 1, H), W2, b2.reshape(E, 1, H), W3,
      b3.reshape(E, 1, D))


_TPW = T // NW


@functools.cache
def _make_sc_combine():
    return functools.partial(
        pl.kernel, mesh=_sc_mesh(),
        out_type=jax.ShapeDtypeStruct((T, D), jnp.float32),
        scratch_types=[
            pltpu.VMEM((_TPW,), jnp.int32),
            pltpu.VMEM((_TPW,), jnp.int32),
            pltpu.VMEM((_TPW, D), jnp.float32),
            pltpu.VMEM((_TPW, D), jnp.float32),
            pltpu.SemaphoreType.DMA,
            pltpu.SemaphoreType.DMA,
        ],
    )(_sc_combine_body)


def _sc_combine(pos1d, ys):
    return _make_sc_combine()(pos1d, ys)


def _sc_combine_body(pos_hbm, ys_hbm, out_hbm,
                     idx0, idx1, buf0, buf1, sem0, sem1):
    wid = lax.axis_index("s") * 2 + lax.axis_index("c")
    base = wid * _TPW
    row = wid // 2
    col = (wid % 2) * _TPW
    pltpu.sync_copy(pos_hbm.at[row, pl.ds(col, _TPW)], idx0)
    pltpu.sync_copy(pos_hbm.at[16 + row, pl.ds(col, _TPW)], idx1)
    cp0 = pltpu.async_copy(ys_hbm.at[idx0], buf0, sem0)
    cp1 = pltpu.async_copy(ys_hbm.at[idx1], buf1, sem1)
    cp0.wait()
    cp1.wait()

    def rbody(r, cr):
        for j in range(D // 16):
            sl = pl.ds(j * 16, 16)
            buf0[r, sl] = buf0[r, sl] + buf1[r, sl]
        return cr

    lax.fori_loop(0, _TPW, rbody, 0)
    pltpu.sync_copy(buf0, out_hbm.at[pl.ds(base, _TPW)])


def kernel(x, Wg, W1, b1, W2, b2, W3, b3):
    B, S, _ = x.shape
    x_flat = x.reshape(T, D)

    pos, tok, prob, be, aux = _router(x_flat, Wg)
    xs, wslot = _sc_dispatch_gather(
        pos, tok, prob,
        (jnp.arange(NPAD, dtype=jnp.int32) * 3 + 128) % T,
        jnp.zeros((NPAD,), jnp.float32), x_flat)
    ys = _gmm(be.reshape(NB + 1), xs, wslot.reshape(NPAD, 1),
              W1, b1, W2, b2, W3, b3)
    out = _sc_combine(pos, ys)
    return out.reshape(B, S, D), aux[0, 0]
